# Initial kernel scaffold; baseline (speedup 1.0000x reference)
import jax
import jax.numpy as jnp
from jax.experimental import pallas as pl


def kernel(x, edge_index, batch, lin_W0, lin_b0, agg_W0, agg_b0, lin_W1, lin_b1, agg_W1, agg_b1, mp_W1, mp_b1, mp_W2, mp_b2):
    raise NotImplementedError("write your pallas kernel here")


# speedup vs baseline: 4.2936x; 4.2936x over previous
import functools

import jax
import jax.numpy as jnp
from jax import lax
from jax.experimental import pallas as pl
from jax.experimental.pallas import tpu as pltpu
from jax.experimental.pallas import tpu_sc as plsc

NC = 2
NS = 16
NW = NC * NS
LANES = 16
CH = 128


def _zero2d(ref, nrow, ncol):
    zv = jnp.zeros((LANES,), jnp.float32)

    def zrow(i, _):
        r = i // (ncol // LANES)
        c = (i % (ncol // LANES)) * LANES
        ref[r, pl.ds(c, LANES)] = zv
        return 0

    lax.fori_loop(0, nrow * (ncol // LANES), zrow, 0)


def _zero_shared(src_v, shared, base, rpt):
    n_full, rem = divmod(rpt, CH)
    for f in range(n_full):
        pltpu.sync_copy(src_v, shared.at[pl.ds(base + f * CH, CH)])
    if rem:
        pltpu.sync_copy(src_v.at[pl.ds(0, rem)],
                        shared.at[pl.ds(base + n_full * CH, rem)])


def _build_sc_sum(Np, K, D):
    RPT = Np // NS
    mesh = plsc.VectorSubcoreMesh(core_axis_name="c", subcore_axis_name="s")

    def body(src_hbm, dst_hbm, table_hbm, sum_hbm,
             idx_s, idx_d, rows_v, accum_sh):
        cid = lax.axis_index("c")
        sid = lax.axis_index("s")
        g = cid * NS + sid
        base = sid * RPT

        _zero2d(rows_v, CH, D)
        _zero_shared(rows_v, accum_sh, base, RPT)
        pltpu.sync_copy(src_hbm.at[g], idx_s)
        pltpu.sync_copy(dst_hbm.at[g], idx_d)

        plsc.subcore_barrier()

        def step(j, _):
            pltpu.sync_copy(table_hbm.at[idx_s.at[j]], rows_v)
            pltpu.sync_copy(rows_v, accum_sh.at[idx_d.at[j]], add=True)
            return 0

        lax.fori_loop(0, K, step, 0)

        plsc.subcore_barrier()

        pltpu.sync_copy(accum_sh.at[pl.ds(base, RPT)],
                        sum_hbm.at[cid, pl.ds(base, RPT)])

    return pl.kernel(
        body,
        out_type=[jax.ShapeDtypeStruct((NC, Np, D), jnp.float32)],
        mesh=mesh,
        scratch_types=[
            pltpu.VMEM((K, CH), jnp.int32),
            pltpu.VMEM((K, CH), jnp.int32),
            pltpu.VMEM((CH, D), jnp.float32),
            pltpu.VMEM_SHARED((Np, D), jnp.float32),
        ],
    )


def _build_sc_cnt(Np, K, CW):
    RPT = Np // NS
    mesh = plsc.VectorSubcoreMesh(core_axis_name="c", subcore_axis_name="s")

    def body(dst_hbm, cnt_hbm, idx_d, ones_v, zc_v, cnt_sh):
        cid = lax.axis_index("c")
        sid = lax.axis_index("s")
        g = cid * NS + sid
        base = sid * RPT

        _zero2d(zc_v, CH, CW)
        ov = jnp.ones((LANES,), jnp.float32)

        def fill(i, _):
            r = i // (CW // LANES)
            c = (i % (CW // LANES)) * LANES
            ones_v[r, pl.ds(c, LANES)] = ov
            return 0

        lax.fori_loop(0, CH * (CW // LANES), fill, 0)
        _zero_shared(zc_v, cnt_sh, base, RPT)
        pltpu.sync_copy(dst_hbm.at[g], idx_d)

        plsc.subcore_barrier()

        def step(j, _):
            pltpu.sync_copy(ones_v, cnt_sh.at[idx_d.at[j]], add=True)
            return 0

        lax.fori_loop(0, K, step, 0)

        plsc.subcore_barrier()

        pltpu.sync_copy(cnt_sh.at[pl.ds(base, RPT)],
                        cnt_hbm.at[cid, pl.ds(base, RPT)])

    return pl.kernel(
        body,
        out_type=[jax.ShapeDtypeStruct((NC, Np, CW), jnp.float32)],
        mesh=mesh,
        scratch_types=[
            pltpu.VMEM((K, CH), jnp.int32),
            pltpu.VMEM((CH, CW), jnp.float32),
            pltpu.VMEM((CH, CW), jnp.float32),
            pltpu.VMEM_SHARED((Np, CW), jnp.float32),
        ],
    )


def _dotT(a, w):
    return lax.dot_general(a, w, (((1,), (1,)), ((), ())),
                           preferred_element_type=jnp.float32)


def _mm_relu(x, W, b, R):
    N, D = x.shape
    H = W.shape[0]

    def body(x_ref, w_ref, b_ref, o_ref):
        o_ref[...] = jnp.maximum(_dotT(x_ref[...], w_ref[...]) + b_ref[...], 0.0)

    return pl.pallas_call(
        body,
        grid=(N // R,),
        in_specs=[pl.BlockSpec((R, D), lambda i: (i, 0)),
                  pl.BlockSpec((H, D), lambda i: (0, 0)),
                  pl.BlockSpec((1, H), lambda i: (0, 0))],
        out_specs=pl.BlockSpec((R, H), lambda i: (i, 0)),
        out_shape=jax.ShapeDtypeStruct((N, H), jnp.float32),
    )(x, W, b.reshape(1, H))


def _sage_update(x_ref, s_ref, c_ref, lw_ref, lb_ref):
    s = s_ref[...]
    c = c_ref[...]
    ssum = s[0] + s[1]
    csum = c[0, :, 0:1] + c[1, :, 0:1]
    mean = ssum / jnp.maximum(csum, 1.0)
    t = jnp.maximum(_dotT(x_ref[...], lw_ref[...]) + lb_ref[...] + mean, 0.0)
    nrm = jnp.sqrt(jnp.sum(t * t, axis=1, keepdims=True))
    return jnp.maximum(t / jnp.maximum(nrm, 1e-12), 0.0)


def _upd_agg(x, sums, cnts, lin_W, lin_b, agg_W, agg_b, R):
    N, D = x.shape
    H = lin_W.shape[0]

    def body(x_ref, s_ref, c_ref, lw_ref, lb_ref, aw_ref, ab_ref,
             h_ref, o_ref):
        h = _sage_update(x_ref, s_ref, c_ref, lw_ref, lb_ref)
        h_ref[...] = h
        o_ref[...] = jnp.maximum(_dotT(h, aw_ref[...]) + ab_ref[...], 0.0)

    return pl.pallas_call(
        body,
        grid=(N // R,),
        in_specs=[pl.BlockSpec((R, D), lambda i: (i, 0)),
                  pl.BlockSpec((NC, R, H), lambda i: (0, i, 0)),
                  pl.BlockSpec((NC, R, H), lambda i: (0, i, 0)),
                  pl.BlockSpec((H, D), lambda i: (0, 0)),
                  pl.BlockSpec((1, H), lambda i: (0, 0)),
                  pl.BlockSpec((H, H), lambda i: (0, 0)),
                  pl.BlockSpec((1, H), lambda i: (0, 0))],
        out_specs=[pl.BlockSpec((R, H), lambda i: (i, 0)),
                   pl.BlockSpec((R, H), lambda i: (i, 0))],
        out_shape=[jax.ShapeDtypeStruct((N, H), jnp.float32),
                   jax.ShapeDtypeStruct((N, H), jnp.float32)],
    )(x, sums, cnts, lin_W, lin_b.reshape(1, H), agg_W, agg_b.reshape(1, H))


def _upd_final(x, sums, cnts, lin_W, lin_b, W1, b1, W2, b2, R):
    N, D = x.shape
    H = lin_W.shape[0]
    O = W2.shape[0]

    def body(x_ref, s_ref, c_ref, lw_ref, lb_ref, w1_ref, b1_ref,
             w2_ref, b2_ref, y_ref):
        h = _sage_update(x_ref, s_ref, c_ref, lw_ref, lb_ref)
        z = _dotT(h, w1_ref[...]) + b1_ref[...]
        z = _dotT(z, w2_ref[...]) + b2_ref[...]
        m = jnp.max(z, axis=1, keepdims=True)
        zs = z - m
        y_ref[...] = zs - jnp.log(jnp.sum(jnp.exp(zs), axis=1, keepdims=True))

    return pl.pallas_call(
        body,
        grid=(N // R,),
        in_specs=[pl.BlockSpec((R, D), lambda i: (i, 0)),
                  pl.BlockSpec((NC, R, H), lambda i: (0, i, 0)),
                  pl.BlockSpec((NC, R, H), lambda i: (0, i, 0)),
                  pl.BlockSpec((H, D), lambda i: (0, 0)),
                  pl.BlockSpec((1, H), lambda i: (0, 0)),
                  pl.BlockSpec((H, H), lambda i: (0, 0)),
                  pl.BlockSpec((1, H), lambda i: (0, 0)),
                  pl.BlockSpec((O, H), lambda i: (0, 0)),
                  pl.BlockSpec((1, O), lambda i: (0, 0))],
        out_specs=pl.BlockSpec((R, O), lambda i: (i, 0)),
        out_shape=jax.ShapeDtypeStruct((N, O), jnp.float32),
    )(x, sums, cnts, lin_W, lin_b.reshape(1, H), W1, b1.reshape(1, H),
      W2, b2.reshape(1, O))


def kernel(x, edge_index, batch, lin_W0, lin_b0, agg_W0, agg_b0,
           lin_W1, lin_b1, agg_W1, agg_b1, mp_W1, mp_b1, mp_W2, mp_b2):
    N, D = x.shape
    E = edge_index.shape[1]
    R = 2000

    E_pad = -(-E // (NW * CH)) * (NW * CH)
    K = E_pad // (NW * CH)
    Np = -(-(N + 1) // (NS * 8)) * (NS * 8)

    src = edge_index[0]
    dst = edge_index[1]
    if E_pad != E:
        pad = E_pad - E
        src = jnp.concatenate([src, jnp.zeros((pad,), jnp.int32)])
        dst = jnp.concatenate([dst, jnp.full((pad,), N, jnp.int32)])
    src_r = src.reshape(NW, K, CH)
    dst_r = dst.reshape(NW, K, CH)

    sc_sum = _build_sc_sum(Np, K, D)
    sc_cnt = _build_sc_cnt(Np, K, D)

    (cnts,) = sc_cnt(dst_r)
    out0 = _mm_relu(x, agg_W0, agg_b0, R)
    (sums0,) = sc_sum(src_r, dst_r, out0)
    h1, out1 = _upd_agg(x, sums0, cnts, lin_W0, lin_b0, agg_W1, agg_b1, R)
    (sums1,) = sc_sum(src_r, dst_r, out1)
    return _upd_final(h1, sums1, cnts, lin_W1, lin_b1,
                      mp_W1, mp_b1, mp_W2, mp_b2, R)
